# baseline (speedup 1.0000x reference)
import functools

import jax
import jax.numpy as jnp
from jax import lax
from jax.experimental import pallas as pl
from jax.experimental.pallas import tpu as pltpu
from jax.experimental.pallas import tpu_sc as plsc

N = 10000
E = 320000
D = 128
L = 6
NC = 2
NS = 16
NW = NC * NS
K = 128
EPW = 10240
NCHUNK = EPW // K
NCB = 8
NBLK = NCHUNK // NCB
NP = 10240
RPS = NP // NS
EPS = 1e-5

_MESH = plsc.VectorSubcoreMesh(
    core_axis_name="c", subcore_axis_name="s", num_cores=NC, num_subcores=NS
)


@functools.partial(
    pl.kernel,
    out_type=jax.ShapeDtypeStruct((NC, NP, D), jnp.float32),
    mesh=_MESH,
    scratch_types=[
        pltpu.VMEM((NCHUNK, K), jnp.int32),
        pltpu.VMEM((K, D), jnp.float32),
        pltpu.VMEM_SHARED((NP, D), jnp.float32),
    ],
)
def _sc_degree(dst_hbm, ones_hbm, zeros_hbm, deg_out, dst_v, ones_v, acc_sh):
    c = lax.axis_index("c")
    s = lax.axis_index("s")
    wid = s * NC + c
    pltpu.sync_copy(dst_hbm.at[wid], dst_v)
    pltpu.sync_copy(ones_hbm, ones_v)
    pltpu.sync_copy(zeros_hbm, acc_sh.at[pl.ds(s * RPS, RPS)])
    plsc.subcore_barrier()

    def chunk_body(j, _):
        pltpu.sync_copy(ones_v, acc_sh.at[dst_v.at[j]], add=True)
        return 0

    lax.fori_loop(0, NCHUNK, chunk_body, 0)
    plsc.subcore_barrier()
    pltpu.sync_copy(acc_sh.at[pl.ds(s * RPS, RPS)],
                    deg_out.at[c, pl.ds(s * RPS, RPS)])


@functools.partial(
    pl.kernel,
    out_type=jax.ShapeDtypeStruct((NC, NP, D), jnp.float32),
    mesh=_MESH,
    scratch_types=[
        pltpu.VMEM((NCHUNK, K), jnp.int32),
        pltpu.VMEM((2 * NCB, K), jnp.int32),
        pltpu.VMEM((K, D), jnp.float32),
        pltpu.VMEM((K, D), jnp.float32),
        pltpu.VMEM_SHARED((NP, D), jnp.float32),
        pltpu.SemaphoreType.DMA,
        pltpu.SemaphoreType.DMA,
    ],
)
def _sc_aggregate(y_hbm, src_hbm, dst_hbm, zeros_hbm, out_hbm,
                  src_v, dstb, rows0, rows1, acc_sh, sg0, sg1):
    c = lax.axis_index("c")
    s = lax.axis_index("s")
    wid = s * NC + c
    pltpu.sync_copy(src_hbm.at[wid], src_v)
    pltpu.sync_copy(dst_hbm.at[wid, pl.ds(0, 2 * NCB)], dstb)
    pltpu.sync_copy(zeros_hbm, acc_sh.at[pl.ds(s * RPS, RPS)])
    plsc.subcore_barrier()

    pltpu.async_copy(y_hbm.at[src_v.at[0]], rows0, sg0)
    pltpu.async_copy(y_hbm.at[src_v.at[1]], rows1, sg1)

    NSB = NCHUNK // (2 * NCB)

    def sb_body(t, _):
        j_base = t * 2 * NCB
        for i in range(2 * NCB):
            rows = rows0 if i % 2 == 0 else rows1
            sg = sg0 if i % 2 == 0 else sg1
            pltpu.make_async_copy(y_hbm.at[src_v.at[0]], rows, sg).wait()
            pltpu.sync_copy(rows, acc_sh.at[dstb.at[i]], add=True)
            pltpu.async_copy(
                y_hbm.at[src_v.at[jnp.minimum(j_base + i + 2, NCHUNK - 1)]],
                rows, sg)
        nb = jnp.minimum((t + 1) * 2 * NCB, NCHUNK - 2 * NCB)
        pltpu.sync_copy(dst_hbm.at[wid, pl.ds(nb, 2 * NCB)], dstb)
        return 0

    lax.fori_loop(0, NSB, sb_body, 0)
    pltpu.make_async_copy(y_hbm.at[src_v.at[0]], rows0, sg0).wait()
    pltpu.make_async_copy(y_hbm.at[src_v.at[0]], rows1, sg1).wait()
    plsc.subcore_barrier()
    pltpu.sync_copy(acc_sh.at[pl.ds(s * RPS, RPS)],
                    out_hbm.at[c, pl.ds(s * RPS, RPS)])


def _bn_relu(z, g, b):
    m = jnp.mean(z, axis=0, keepdims=True)
    v = jnp.mean((z - m) * (z - m), axis=0, keepdims=True)
    return jnp.maximum((z - m) * jax.lax.rsqrt(v + EPS) * g + b, 0.0)


def _tc_input_body(x_ref, inW_ref, inb_ref, ing_ref, inbeta_ref, degp_ref,
                   W0_ref, h_ref, y_ref, dinv_ref):
    z = jnp.dot(x_ref[...], inW_ref[...],
                preferred_element_type=jnp.float32,
                precision=jax.lax.Precision.HIGHEST) + inb_ref[...]
    h = _bn_relu(z, ing_ref[...], inbeta_ref[...])
    h_ref[...] = h
    deg = degp_ref[0, :N, 0:1] + degp_ref[1, :N, 0:1] + 1.0
    dinv = jax.lax.rsqrt(deg)
    dinv_ref[...] = dinv
    y_ref[...] = jnp.dot(h, W0_ref[...],
                         preferred_element_type=jnp.float32,
                         precision=jax.lax.Precision.HIGHEST) * dinv


def _tc_layer_body(acc_ref, y_ref, h_ref, dinv_ref, cb_ref, g_ref, b_ref,
                   Wn_ref, hn_ref, yn_ref):
    dinv = dinv_ref[...]
    z = (acc_ref[0, :N] + acc_ref[1, :N] + y_ref[...]) * dinv + cb_ref[...]
    hn = _bn_relu(z, g_ref[...], b_ref[...]) + h_ref[...]
    hn_ref[...] = hn
    yn_ref[...] = jnp.dot(hn, Wn_ref[...],
                          preferred_element_type=jnp.float32,
                          precision=jax.lax.Precision.HIGHEST) * dinv


def _tc_final_body(acc_ref, y_ref, h_ref, dinv_ref, cb_ref, g_ref, b_ref,
                   pW1_ref, pb1_ref, pW2_ref, pb2_ref,
                   vW1_ref, vb1_ref, vW2_ref, vb2_ref,
                   pol_ref, val_ref):
    z = (acc_ref[0, :N] + acc_ref[1, :N] + y_ref[...]) * dinv_ref[...] + cb_ref[...]
    hn = _bn_relu(z, g_ref[...], b_ref[...]) + h_ref[...]
    hp = jnp.maximum(
        jnp.dot(hn, pW1_ref[...], preferred_element_type=jnp.float32,
                precision=jax.lax.Precision.HIGHEST) + pb1_ref[...], 0.0)
    pol_ref[...] = jnp.dot(hp, pW2_ref[...],
                           preferred_element_type=jnp.float32,
                           precision=jax.lax.Precision.HIGHEST) + pb2_ref[...]
    gstate = jnp.mean(hn, axis=0, keepdims=True)
    hv = jnp.maximum(
        jnp.dot(gstate, vW1_ref[...], preferred_element_type=jnp.float32,
                precision=jax.lax.Precision.HIGHEST) + vb1_ref[...], 0.0)
    val_ref[...] = jnp.tanh(
        jnp.dot(hv, vW2_ref[...], preferred_element_type=jnp.float32,
                precision=jax.lax.Precision.HIGHEST) + vb2_ref[...])


_f32 = jnp.float32
_TC_PARAMS = pltpu.CompilerParams(vmem_limit_bytes=100 * 1024 * 1024)

_tc_input = pl.pallas_call(
    _tc_input_body,
    compiler_params=_TC_PARAMS,
    out_shape=[
        jax.ShapeDtypeStruct((N, D), _f32),
        jax.ShapeDtypeStruct((N, D), _f32),
        jax.ShapeDtypeStruct((N, 1), _f32),
    ],
)

_tc_layer = pl.pallas_call(
    _tc_layer_body,
    compiler_params=_TC_PARAMS,
    out_shape=[
        jax.ShapeDtypeStruct((N, D), _f32),
        jax.ShapeDtypeStruct((N, D), _f32),
    ],
)

_tc_final = pl.pallas_call(
    _tc_final_body,
    compiler_params=_TC_PARAMS,
    out_shape=[
        jax.ShapeDtypeStruct((N, 1), _f32),
        jax.ShapeDtypeStruct((1, 1), _f32),
    ],
)


def kernel(x, edge_index, in_W, in_b, in_g, in_beta, conv_W, conv_b,
           bn_g, bn_b, p_W1, p_b1, p_W2, p_b2, v_W1, v_b1, v_W2, v_b2):
    pad = NW * EPW - E
    srcp = jnp.concatenate([edge_index[0], jnp.zeros((pad,), jnp.int32)])
    dstp = jnp.concatenate([edge_index[1], jnp.full((pad,), NP - 1, jnp.int32)])
    src2d = srcp.reshape(NW, NCHUNK, K)
    dst2d = dstp.reshape(NW, NCHUNK, K)
    zeros = jnp.zeros((RPS, D), _f32)
    onesKD = jnp.ones((K, D), _f32)

    deg_parts = _sc_degree(dst2d, onesKD, zeros)
    h, y, dinv = _tc_input(x, in_W, in_b.reshape(1, D), in_g.reshape(1, D),
                           in_beta.reshape(1, D), deg_parts, conv_W[0])
    for i in range(L - 1):
        acc = _sc_aggregate(y, src2d, dst2d, zeros)
        h, y = _tc_layer(acc, y, h, dinv, conv_b[i].reshape(1, D),
                         bn_g[i].reshape(1, D), bn_b[i].reshape(1, D),
                         conv_W[i + 1])
    acc = _sc_aggregate(y, src2d, dst2d, zeros)
    pol, val = _tc_final(acc, y, h, dinv, conv_b[L - 1].reshape(1, D),
                         bn_g[L - 1].reshape(1, D), bn_b[L - 1].reshape(1, D),
                         p_W1, p_b1.reshape(1, 32), p_W2, p_b2.reshape(1, 1),
                         v_W1, v_b1.reshape(1, 64), v_W2, v_b2.reshape(1, 1))
    return (pol.reshape(N), val.reshape(1))


# speedup vs baseline: 3.1279x; 3.1279x over previous
import functools

import jax
import jax.numpy as jnp
from jax import lax
from jax.experimental import pallas as pl
from jax.experimental.pallas import tpu as pltpu
from jax.experimental.pallas import tpu_sc as plsc

N = 10000
E = 320000
D = 128
L = 6
NC = 2
NS = 16
NW = NC * NS
K = 128
EPW = 10240
NCHUNK = EPW // K
NCB = 8
NBLK = NCHUNK // NCB
NP = 10240
RPS = NP // NS
EPS = 1e-5

_MESH = plsc.VectorSubcoreMesh(
    core_axis_name="c", subcore_axis_name="s", num_cores=NC, num_subcores=NS
)


@functools.partial(
    pl.kernel,
    out_type=jax.ShapeDtypeStruct((NC, NP, D), jnp.float32),
    mesh=_MESH,
    scratch_types=[
        pltpu.VMEM((NCHUNK, K), jnp.int32),
        pltpu.VMEM((K, D), jnp.float32),
        pltpu.VMEM_SHARED((NP, D), jnp.float32),
    ],
)
def _sc_degree(dst_hbm, ones_hbm, zeros_hbm, deg_out, dst_v, ones_v, acc_sh):
    c = lax.axis_index("c")
    s = lax.axis_index("s")
    wid = s * NC + c
    pltpu.sync_copy(dst_hbm.at[wid], dst_v)
    pltpu.sync_copy(ones_hbm, ones_v)
    pltpu.sync_copy(zeros_hbm, acc_sh.at[pl.ds(s * RPS, RPS)])
    plsc.subcore_barrier()

    def chunk_body(j, _):
        pltpu.sync_copy(ones_v, acc_sh.at[dst_v.at[j]], add=True)
        return 0

    lax.fori_loop(0, NCHUNK, chunk_body, 0)
    plsc.subcore_barrier()
    pltpu.sync_copy(acc_sh.at[pl.ds(s * RPS, RPS)],
                    deg_out.at[c, pl.ds(s * RPS, RPS)])


@functools.partial(
    pl.kernel,
    out_type=jax.ShapeDtypeStruct((NC, NP, D), jnp.float32),
    mesh=_MESH,
    scratch_types=[
        pltpu.VMEM((NCHUNK, K), jnp.int32),
        pltpu.VMEM((2 * NCB, K), jnp.int32),
        pltpu.VMEM((K, D), jnp.float32),
        pltpu.VMEM((K, D), jnp.float32),
        pltpu.VMEM_SHARED((NP, D), jnp.float32),
        pltpu.SemaphoreType.DMA,
        pltpu.SemaphoreType.DMA,
    ],
)
def _sc_aggregate(y_hbm, src_hbm, dst_hbm, zeros_hbm, out_hbm,
                  src_v, dstb, rows0, rows1, acc_sh, sg0, sg1):
    c = lax.axis_index("c")
    s = lax.axis_index("s")
    wid = s * NC + c
    pltpu.sync_copy(src_hbm.at[wid], src_v)
    pltpu.sync_copy(dst_hbm.at[wid, pl.ds(0, 2 * NCB)], dstb)
    pltpu.sync_copy(zeros_hbm, acc_sh.at[pl.ds(s * RPS, RPS)])
    plsc.subcore_barrier()

    pltpu.async_copy(y_hbm.at[src_v.at[0]], rows0, sg0)
    pltpu.async_copy(y_hbm.at[src_v.at[1]], rows1, sg1)

    NSB = NCHUNK // (2 * NCB)

    def sb_body(t, _):
        j_base = t * 2 * NCB
        for i in range(2 * NCB):
            rows = rows0 if i % 2 == 0 else rows1
            sg = sg0 if i % 2 == 0 else sg1
            pltpu.make_async_copy(y_hbm.at[src_v.at[0]], rows, sg).wait()
            pltpu.sync_copy(rows, acc_sh.at[dstb.at[i]], add=True)
            pltpu.async_copy(
                y_hbm.at[src_v.at[jnp.minimum(j_base + i + 2, NCHUNK - 1)]],
                rows, sg)
        nb = jnp.minimum((t + 1) * 2 * NCB, NCHUNK - 2 * NCB)
        pltpu.sync_copy(dst_hbm.at[wid, pl.ds(nb, 2 * NCB)], dstb)
        return 0

    lax.fori_loop(0, NSB, sb_body, 0)
    pltpu.make_async_copy(y_hbm.at[src_v.at[0]], rows0, sg0).wait()
    pltpu.make_async_copy(y_hbm.at[src_v.at[0]], rows1, sg1).wait()
    plsc.subcore_barrier()
    pltpu.sync_copy(acc_sh.at[pl.ds(s * RPS, RPS)],
                    out_hbm.at[c, pl.ds(s * RPS, RPS)])


def _bn_relu(z, g, b):
    m = jnp.mean(z, axis=0, keepdims=True)
    v = jnp.mean((z - m) * (z - m), axis=0, keepdims=True)
    return jnp.maximum((z - m) * jax.lax.rsqrt(v + EPS) * g + b, 0.0)


def _tc_input_body(x_ref, inW_ref, inb_ref, ing_ref, inbeta_ref, degp_ref,
                   W0_ref, h_ref, y_ref, dinv_ref):
    z = jnp.dot(x_ref[...], inW_ref[...],
                preferred_element_type=jnp.float32,
                precision=jax.lax.Precision.HIGHEST) + inb_ref[...]
    h = _bn_relu(z, ing_ref[...], inbeta_ref[...])
    h_ref[...] = h
    deg = degp_ref[0, :N, 0:1] + degp_ref[1, :N, 0:1] + 1.0
    dinv = jax.lax.rsqrt(deg)
    dinv_ref[...] = dinv
    y_ref[...] = jnp.dot(h, W0_ref[...],
                         preferred_element_type=jnp.float32,
                         precision=jax.lax.Precision.HIGHEST) * dinv


def _tc_layer_body(acc_ref, y_ref, h_ref, dinv_ref, cb_ref, g_ref, b_ref,
                   Wn_ref, hn_ref, yn_ref):
    dinv = dinv_ref[...]
    z = (acc_ref[0, :N] + acc_ref[1, :N] + y_ref[...]) * dinv + cb_ref[...]
    hn = _bn_relu(z, g_ref[...], b_ref[...]) + h_ref[...]
    hn_ref[...] = hn
    yn_ref[...] = jnp.dot(hn, Wn_ref[...],
                          preferred_element_type=jnp.float32,
                          precision=jax.lax.Precision.HIGHEST) * dinv


def _tc_final_body(acc_ref, y_ref, h_ref, dinv_ref, cb_ref, g_ref, b_ref,
                   pW1_ref, pb1_ref, pW2_ref, pb2_ref,
                   vW1_ref, vb1_ref, vW2_ref, vb2_ref,
                   pol_ref, val_ref):
    z = (acc_ref[0, :N] + acc_ref[1, :N] + y_ref[...]) * dinv_ref[...] + cb_ref[...]
    hn = _bn_relu(z, g_ref[...], b_ref[...]) + h_ref[...]
    hp = jnp.maximum(
        jnp.dot(hn, pW1_ref[...], preferred_element_type=jnp.float32,
                precision=jax.lax.Precision.HIGHEST) + pb1_ref[...], 0.0)
    pol_ref[...] = jnp.dot(hp, pW2_ref[...],
                           preferred_element_type=jnp.float32,
                           precision=jax.lax.Precision.HIGHEST) + pb2_ref[...]
    gstate = jnp.mean(hn, axis=0, keepdims=True)
    hv = jnp.maximum(
        jnp.dot(gstate, vW1_ref[...], preferred_element_type=jnp.float32,
                precision=jax.lax.Precision.HIGHEST) + vb1_ref[...], 0.0)
    val_ref[...] = jnp.tanh(
        jnp.dot(hv, vW2_ref[...], preferred_element_type=jnp.float32,
                precision=jax.lax.Precision.HIGHEST) + vb2_ref[...])


_f32 = jnp.float32
_TC_PARAMS = pltpu.CompilerParams(vmem_limit_bytes=100 * 1024 * 1024)

_tc_input = pl.pallas_call(
    _tc_input_body,
    compiler_params=_TC_PARAMS,
    out_shape=[
        jax.ShapeDtypeStruct((N, D), _f32),
        jax.ShapeDtypeStruct((N, D), _f32),
        jax.ShapeDtypeStruct((N, 1), _f32),
    ],
)

_tc_layer = pl.pallas_call(
    _tc_layer_body,
    compiler_params=_TC_PARAMS,
    out_shape=[
        jax.ShapeDtypeStruct((N, D), _f32),
        jax.ShapeDtypeStruct((N, D), _f32),
    ],
)

_tc_final = pl.pallas_call(
    _tc_final_body,
    compiler_params=_TC_PARAMS,
    out_shape=[
        jax.ShapeDtypeStruct((N, 1), _f32),
        jax.ShapeDtypeStruct((1, 1), _f32),
    ],
)


def kernel(x, edge_index, in_W, in_b, in_g, in_beta, conv_W, conv_b,
           bn_g, bn_b, p_W1, p_b1, p_W2, p_b2, v_W1, v_b1, v_W2, v_b2):
    pad = NW * EPW - E
    pi = jnp.arange(pad, dtype=jnp.int32)
    srcp = jnp.concatenate([edge_index[0], pi % N])
    dstp = jnp.concatenate([edge_index[1], N + pi % (NP - N)])
    src2d = srcp.reshape(NW, NCHUNK, K)
    dst2d = dstp.reshape(NW, NCHUNK, K)
    zeros = jnp.zeros((RPS, D), _f32)
    onesKD = jnp.ones((K, D), _f32)

    deg_parts = _sc_degree(dst2d, onesKD, zeros)
    h, y, dinv = _tc_input(x, in_W, in_b.reshape(1, D), in_g.reshape(1, D),
                           in_beta.reshape(1, D), deg_parts, conv_W[0])
    for i in range(L - 1):
        acc = _sc_aggregate(y, src2d, dst2d, zeros)
        h, y = _tc_layer(acc, y, h, dinv, conv_b[i].reshape(1, D),
                         bn_g[i].reshape(1, D), bn_b[i].reshape(1, D),
                         conv_W[i + 1])
    acc = _sc_aggregate(y, src2d, dst2d, zeros)
    pol, val = _tc_final(acc, y, h, dinv, conv_b[L - 1].reshape(1, D),
                         bn_g[L - 1].reshape(1, D), bn_b[L - 1].reshape(1, D),
                         p_W1, p_b1.reshape(1, 32), p_W2, p_b2.reshape(1, 1),
                         v_W1, v_b1.reshape(1, 64), v_W2, v_b2.reshape(1, 1))
    return (pol.reshape(N), val.reshape(1))


# speedup vs baseline: 3.2094x; 1.0261x over previous
import functools

import jax
import jax.numpy as jnp
from jax import lax
from jax.experimental import pallas as pl
from jax.experimental.pallas import tpu as pltpu
from jax.experimental.pallas import tpu_sc as plsc

N = 10000
E = 320000
D = 128
L = 6
NC = 2
NS = 16
NW = NC * NS
K = 128
EPW = 10240
NCHUNK = EPW // K
NCB = 8
NBLK = NCHUNK // NCB
NP = 10240
RPS = NP // NS
EPS = 1e-5

_MESH = plsc.VectorSubcoreMesh(
    core_axis_name="c", subcore_axis_name="s", num_cores=NC, num_subcores=NS
)


@functools.partial(
    pl.kernel,
    out_type=jax.ShapeDtypeStruct((NC, NP, D), jnp.float32),
    mesh=_MESH,
    scratch_types=[
        pltpu.VMEM((NCHUNK, K), jnp.int32),
        pltpu.VMEM((K, D), jnp.float32),
        pltpu.VMEM_SHARED((NP, D), jnp.float32),
    ],
)
def _sc_degree(dst_hbm, ones_hbm, zeros_hbm, deg_out, dst_v, ones_v, acc_sh):
    c = lax.axis_index("c")
    s = lax.axis_index("s")
    wid = s * NC + c
    pltpu.sync_copy(dst_hbm.at[wid], dst_v)
    pltpu.sync_copy(ones_hbm, ones_v)
    pltpu.sync_copy(zeros_hbm, acc_sh.at[pl.ds(s * RPS, RPS)])
    plsc.subcore_barrier()

    def chunk_body(j, _):
        pltpu.sync_copy(ones_v, acc_sh.at[dst_v.at[j]], add=True)
        return 0

    lax.fori_loop(0, NCHUNK, chunk_body, 0)
    plsc.subcore_barrier()
    pltpu.sync_copy(acc_sh.at[pl.ds(s * RPS, RPS)],
                    deg_out.at[c, pl.ds(s * RPS, RPS)])


@functools.partial(
    pl.kernel,
    out_type=jax.ShapeDtypeStruct((NC, NP, D), jnp.float32),
    mesh=_MESH,
    scratch_types=[
        pltpu.VMEM((NCHUNK, K), jnp.int32),
        pltpu.VMEM((2 * NCB, K), jnp.int32),
        pltpu.VMEM((K, D), jnp.float32),
        pltpu.VMEM((K, D), jnp.float32),
        pltpu.VMEM_SHARED((NP, D), jnp.float32),
        pltpu.SemaphoreType.DMA,
        pltpu.SemaphoreType.DMA,
    ],
)
def _sc_aggregate(y_hbm, src_hbm, dst_hbm, zeros_hbm, out_hbm,
                  src_v, dstb, rows0, rows1, acc_sh, sg0, sg1):
    c = lax.axis_index("c")
    s = lax.axis_index("s")
    wid = s * NC + c
    pltpu.sync_copy(src_hbm.at[wid], src_v)
    pltpu.sync_copy(dst_hbm.at[wid, pl.ds(0, 2 * NCB)], dstb)
    pltpu.sync_copy(zeros_hbm, acc_sh.at[pl.ds(s * RPS, RPS)])
    pltpu.async_copy(y_hbm.at[src_v.at[0]], rows0, sg0)
    pltpu.async_copy(y_hbm.at[src_v.at[1]], rows1, sg1)
    plsc.subcore_barrier()

    NSB = NCHUNK // (2 * NCB)

    def sb_body(t, _):
        j_base = t * 2 * NCB
        for i in range(2 * NCB):
            rows = rows0 if i % 2 == 0 else rows1
            sg = sg0 if i % 2 == 0 else sg1
            pltpu.make_async_copy(y_hbm.at[src_v.at[0]], rows, sg).wait()
            pltpu.sync_copy(rows, acc_sh.at[dstb.at[i]], add=True)
            pltpu.async_copy(
                y_hbm.at[src_v.at[jnp.minimum(j_base + i + 2, NCHUNK - 1)]],
                rows, sg)
        nb = jnp.minimum((t + 1) * 2 * NCB, NCHUNK - 2 * NCB)
        pltpu.sync_copy(dst_hbm.at[wid, pl.ds(nb, 2 * NCB)], dstb)
        return 0

    lax.fori_loop(0, NSB, sb_body, 0)
    pltpu.make_async_copy(y_hbm.at[src_v.at[0]], rows0, sg0).wait()
    pltpu.make_async_copy(y_hbm.at[src_v.at[0]], rows1, sg1).wait()
    plsc.subcore_barrier()
    pltpu.sync_copy(acc_sh.at[pl.ds(s * RPS, RPS)],
                    out_hbm.at[c, pl.ds(s * RPS, RPS)])


def _bn_relu(z, g, b):
    m = jnp.mean(z, axis=0, keepdims=True)
    v = jnp.mean((z - m) * (z - m), axis=0, keepdims=True)
    return jnp.maximum((z - m) * jax.lax.rsqrt(v + EPS) * g + b, 0.0)


def _tc_pre_body(x_ref, inW_ref, inb_ref, ing_ref, inbeta_ref,
                 W0_ref, h_ref, xw_ref):
    z = jnp.dot(x_ref[...], inW_ref[...],
                preferred_element_type=jnp.float32,
                precision=jax.lax.Precision.HIGHEST) + inb_ref[...]
    h = _bn_relu(z, ing_ref[...], inbeta_ref[...])
    h_ref[...] = h
    xw_ref[...] = jnp.dot(h, W0_ref[...],
                          preferred_element_type=jnp.float32,
                          precision=jax.lax.Precision.HIGHEST)


def _tc_scale_body(xw_ref, degp_ref, y_ref, dinv_ref):
    deg = degp_ref[0, :N, 0:1] + degp_ref[1, :N, 0:1] + 1.0
    dinv = jax.lax.rsqrt(deg)
    dinv_ref[...] = dinv
    y_ref[...] = xw_ref[...] * dinv


def _tc_layer_body(acc_ref, y_ref, h_ref, dinv_ref, cb_ref, g_ref, b_ref,
                   Wn_ref, hn_ref, yn_ref):
    dinv = dinv_ref[...]
    z = (acc_ref[0, :N] + acc_ref[1, :N] + y_ref[...]) * dinv + cb_ref[...]
    hn = _bn_relu(z, g_ref[...], b_ref[...]) + h_ref[...]
    hn_ref[...] = hn
    yn_ref[...] = jnp.dot(hn, Wn_ref[...],
                          preferred_element_type=jnp.float32,
                          precision=jax.lax.Precision.HIGHEST) * dinv


def _tc_final_body(acc_ref, y_ref, h_ref, dinv_ref, cb_ref, g_ref, b_ref,
                   pW1_ref, pb1_ref, pW2_ref, pb2_ref,
                   vW1_ref, vb1_ref, vW2_ref, vb2_ref,
                   pol_ref, val_ref):
    z = (acc_ref[0, :N] + acc_ref[1, :N] + y_ref[...]) * dinv_ref[...] + cb_ref[...]
    hn = _bn_relu(z, g_ref[...], b_ref[...]) + h_ref[...]
    hp = jnp.maximum(
        jnp.dot(hn, pW1_ref[...], preferred_element_type=jnp.float32,
                precision=jax.lax.Precision.HIGHEST) + pb1_ref[...], 0.0)
    pol_ref[...] = jnp.dot(hp, pW2_ref[...],
                           preferred_element_type=jnp.float32,
                           precision=jax.lax.Precision.HIGHEST) + pb2_ref[...]
    gstate = jnp.mean(hn, axis=0, keepdims=True)
    hv = jnp.maximum(
        jnp.dot(gstate, vW1_ref[...], preferred_element_type=jnp.float32,
                precision=jax.lax.Precision.HIGHEST) + vb1_ref[...], 0.0)
    val_ref[...] = jnp.tanh(
        jnp.dot(hv, vW2_ref[...], preferred_element_type=jnp.float32,
                precision=jax.lax.Precision.HIGHEST) + vb2_ref[...])


_f32 = jnp.float32
_TC_PARAMS = pltpu.CompilerParams(vmem_limit_bytes=100 * 1024 * 1024)

_tc_pre = pl.pallas_call(
    _tc_pre_body,
    compiler_params=_TC_PARAMS,
    out_shape=[
        jax.ShapeDtypeStruct((N, D), _f32),
        jax.ShapeDtypeStruct((N, D), _f32),
    ],
)

_tc_scale = pl.pallas_call(
    _tc_scale_body,
    compiler_params=_TC_PARAMS,
    out_shape=[
        jax.ShapeDtypeStruct((N, D), _f32),
        jax.ShapeDtypeStruct((N, 1), _f32),
    ],
)

_tc_layer = pl.pallas_call(
    _tc_layer_body,
    compiler_params=_TC_PARAMS,
    out_shape=[
        jax.ShapeDtypeStruct((N, D), _f32),
        jax.ShapeDtypeStruct((N, D), _f32),
    ],
)

_tc_final = pl.pallas_call(
    _tc_final_body,
    compiler_params=_TC_PARAMS,
    out_shape=[
        jax.ShapeDtypeStruct((N, 1), _f32),
        jax.ShapeDtypeStruct((1, 1), _f32),
    ],
)


def kernel(x, edge_index, in_W, in_b, in_g, in_beta, conv_W, conv_b,
           bn_g, bn_b, p_W1, p_b1, p_W2, p_b2, v_W1, v_b1, v_W2, v_b2):
    pad = NW * EPW - E
    pi = jnp.arange(pad, dtype=jnp.int32)
    srcp = jnp.concatenate([edge_index[0], pi % N])
    dstp = jnp.concatenate([edge_index[1], N + pi % (NP - N)])
    src2d = srcp.reshape(NW, NCHUNK, K)
    dst2d = dstp.reshape(NW, NCHUNK, K)
    zeros = jnp.zeros((RPS, D), _f32)
    onesKD = jnp.ones((K, D), _f32)

    deg_parts = _sc_degree(dst2d, onesKD, zeros)
    h, xw = _tc_pre(x, in_W, in_b.reshape(1, D), in_g.reshape(1, D),
                    in_beta.reshape(1, D), conv_W[0])
    y, dinv = _tc_scale(xw, deg_parts)
    for i in range(L - 1):
        acc = _sc_aggregate(y, src2d, dst2d, zeros)
        h, y = _tc_layer(acc, y, h, dinv, conv_b[i].reshape(1, D),
                         bn_g[i].reshape(1, D), bn_b[i].reshape(1, D),
                         conv_W[i + 1])
    acc = _sc_aggregate(y, src2d, dst2d, zeros)
    pol, val = _tc_final(acc, y, h, dinv, conv_b[L - 1].reshape(1, D),
                         bn_g[L - 1].reshape(1, D), bn_b[L - 1].reshape(1, D),
                         p_W1, p_b1.reshape(1, 32), p_W2, p_b2.reshape(1, 1),
                         v_W1, v_b1.reshape(1, 64), v_W2, v_b2.reshape(1, 1))
    return (pol.reshape(N), val.reshape(1))
